# baseline (speedup 1.0000x reference)
import jax
import jax.numpy as jnp
from jax import lax
from jax.experimental import pallas as pl
from jax.experimental.pallas import tpu as pltpu
from jax.experimental.pallas import tpu_sc as plsc

_N_ENT = 100000
_N_REL = 1000
_D = 64

_N_ENT_V = _N_ENT // 2
_D_V = 2 * _D
_S = 4
_CH_TC = 2500
_NCH_TC = _N_ENT_V // _CH_TC


def _tc_ent_body(a_hbm, b_hbm, o_hbm, ab, bb, ob, insem, outsem):
    def in_copies(k, s):
        rows = pl.ds(k * _CH_TC, _CH_TC)
        return (
            pltpu.make_async_copy(a_hbm.at[rows], ab.at[s], insem.at[s]),
            pltpu.make_async_copy(b_hbm.at[rows], bb.at[s], insem.at[s]),
        )

    def out_copy(k, s):
        rows = pl.ds(k * _CH_TC, _CH_TC)
        return pltpu.make_async_copy(ob.at[s], o_hbm.at[rows], outsem.at[s])

    for s in range(_S):
        for c in in_copies(s, s):
            c.start()

    def step(k, _):
        s = lax.rem(k, _S)
        for c in in_copies(k, s):
            c.wait()

        @pl.when(k >= _S)
        def _():
            out_copy(k - _S, s).wait()

        ob[s] = ab[s] + bb[s]
        out_copy(k, s).start()

        @pl.when(k + _S < _NCH_TC)
        def _():
            for c in in_copies(k + _S, s):
                c.start()

        return 0

    lax.fori_loop(0, _NCH_TC, step, 0)

    for s in range(_S):
        out_copy(_NCH_TC - _S + s, (_NCH_TC - _S + s) % _S).wait()


_tc_ent_add = pl.pallas_call(
    _tc_ent_body,
    in_specs=[
        pl.BlockSpec(memory_space=pl.ANY),
        pl.BlockSpec(memory_space=pl.ANY),
    ],
    out_specs=pl.BlockSpec(memory_space=pl.ANY),
    out_shape=jax.ShapeDtypeStruct((_N_ENT_V, _D_V), jnp.float32),
    scratch_shapes=[
        pltpu.VMEM((_S, _CH_TC, _D_V), jnp.float32),
        pltpu.VMEM((_S, _CH_TC, _D_V), jnp.float32),
        pltpu.VMEM((_S, _CH_TC, _D_V), jnp.float32),
        pltpu.SemaphoreType.DMA((_S,)),
        pltpu.SemaphoreType.DMA((_S,)),
    ],
)

_NUM_CORES = 2
_NUM_SUBCORES = 16
_LANES = 16
_RCH = 32
_REL_LAST = _N_REL - _RCH


def _add_rows(a, b, rows, rpi):
    def step(i, _):
        r0 = i * rpi
        for r in range(rpi):
            for j in range(_D // _LANES):
                sl = pl.ds(j * _LANES, _LANES)
                a[r0 + r, sl] = a[r0 + r, sl] + b[r0 + r, sl]
        return 0

    lax.fori_loop(0, rows // rpi, step, 0)


def _sc_rel_body(r0, r1, out_r, a, b):
    wid = lax.axis_index("s") * _NUM_CORES + lax.axis_index("c")
    base = pl.multiple_of(jnp.minimum(wid * _RCH, _REL_LAST), 8)
    rows = pl.ds(base, _RCH)
    pltpu.sync_copy(r0.at[rows], a)
    pltpu.sync_copy(r1.at[rows], b)
    _add_rows(a, b, _RCH, 8)
    pltpu.sync_copy(a, out_r.at[rows])


_sc_rel_add = pl.kernel(
    _sc_rel_body,
    out_type=jax.ShapeDtypeStruct((_N_REL, _D), jnp.float32),
    mesh=plsc.VectorSubcoreMesh(
        core_axis_name="c",
        subcore_axis_name="s",
        num_cores=_NUM_CORES,
        num_subcores=_NUM_SUBCORES,
    ),
    scratch_types=[
        pltpu.VMEM((_RCH, _D), jnp.float32),
        pltpu.VMEM((_RCH, _D), jnp.float32),
    ],
)


_BLK = 5000


def _tc_both_body(a, b, r0, r1, o, orel):
    o[...] = a[...] + b[...]

    @pl.when(pl.program_id(0) == 0)
    def _():
        orel[...] = r0[...] + r1[...]


_tc_both_add = pl.pallas_call(
    _tc_both_body,
    grid=(_N_ENT // _BLK,),
    in_specs=[
        pl.BlockSpec((_BLK, _D), lambda i: (i, 0)),
        pl.BlockSpec((_BLK, _D), lambda i: (i, 0)),
        pl.BlockSpec((_N_REL, _D), lambda i: (0, 0)),
        pl.BlockSpec((_N_REL, _D), lambda i: (0, 0)),
    ],
    out_specs=[
        pl.BlockSpec((_BLK, _D), lambda i: (i, 0)),
        pl.BlockSpec((_N_REL, _D), lambda i: (0, 0)),
    ],
    out_shape=[
        jax.ShapeDtypeStruct((_N_ENT, _D), jnp.float32),
        jax.ShapeDtypeStruct((_N_REL, _D), jnp.float32),
    ],
)


def kernel(inputs, ent_embeds_0, rel_embeds_0, ent_embeds_1, rel_embeds_1):
    del inputs
    e0 = ent_embeds_0.reshape(_N_ENT_V, _D_V)
    e1 = ent_embeds_1.reshape(_N_ENT_V, _D_V)
    out_ent = _tc_ent_add(e0, e1).reshape(_N_ENT, _D)
    out_rel = _sc_rel_add(rel_embeds_0, rel_embeds_1)
    return (out_ent, out_rel)


# speedup vs baseline: 1.3545x; 1.3545x over previous
import jax
import jax.numpy as jnp
from jax import lax
from jax.experimental import pallas as pl
from jax.experimental.pallas import tpu as pltpu
from jax.experimental.pallas import tpu_sc as plsc

_N_ENT = 100000
_N_REL = 1000
_D = 64

_N_ENT_V = _N_ENT // 2
_D_V = 2 * _D
_S = 4
_CH_TC = 2500
_NCH_TC = _N_ENT_V // _CH_TC


def _tc_ent_body(a_hbm, b_hbm, o_hbm, ab, bb, ob, insem, outsem):
    def in_copies(k, s):
        rows = pl.ds(k * _CH_TC, _CH_TC)
        return (
            pltpu.make_async_copy(a_hbm.at[rows], ab.at[s], insem.at[s]),
            pltpu.make_async_copy(b_hbm.at[rows], bb.at[s], insem.at[s]),
        )

    def out_copy(k, s):
        rows = pl.ds(k * _CH_TC, _CH_TC)
        return pltpu.make_async_copy(ob.at[s], o_hbm.at[rows], outsem.at[s])

    for s in range(_S):
        for c in in_copies(s, s):
            c.start()

    def step(k, _):
        s = lax.rem(k, _S)
        for c in in_copies(k, s):
            c.wait()

        @pl.when(k >= _S)
        def _():
            out_copy(k - _S, s).wait()

        ob[s] = ab[s] + bb[s]
        out_copy(k, s).start()

        @pl.when(k + _S < _NCH_TC)
        def _():
            for c in in_copies(k + _S, s):
                c.start()

        return 0

    lax.fori_loop(0, _NCH_TC, step, 0)

    for s in range(_S):
        out_copy(_NCH_TC - _S + s, (_NCH_TC - _S + s) % _S).wait()


_tc_ent_add = pl.pallas_call(
    _tc_ent_body,
    in_specs=[
        pl.BlockSpec(memory_space=pl.ANY),
        pl.BlockSpec(memory_space=pl.ANY),
    ],
    out_specs=pl.BlockSpec(memory_space=pl.ANY),
    out_shape=jax.ShapeDtypeStruct((_N_ENT_V, _D_V), jnp.float32),
    scratch_shapes=[
        pltpu.VMEM((_S, _CH_TC, _D_V), jnp.float32),
        pltpu.VMEM((_S, _CH_TC, _D_V), jnp.float32),
        pltpu.VMEM((_S, _CH_TC, _D_V), jnp.float32),
        pltpu.SemaphoreType.DMA((_S,)),
        pltpu.SemaphoreType.DMA((_S,)),
    ],
)

_NUM_CORES = 2
_NUM_SUBCORES = 16
_LANES = 16
_RCH = 32
_REL_LAST = _N_REL - _RCH


def _add_rows(a, b, rows, rpi):
    def step(i, _):
        r0 = i * rpi
        for r in range(rpi):
            for j in range(_D // _LANES):
                sl = pl.ds(j * _LANES, _LANES)
                a[r0 + r, sl] = a[r0 + r, sl] + b[r0 + r, sl]
        return 0

    lax.fori_loop(0, rows // rpi, step, 0)


def _sc_rel_body(r0, r1, out_r, a, b):
    wid = lax.axis_index("s") * _NUM_CORES + lax.axis_index("c")
    base = pl.multiple_of(jnp.minimum(wid * _RCH, _REL_LAST), 8)
    rows = pl.ds(base, _RCH)
    pltpu.sync_copy(r0.at[rows], a)
    pltpu.sync_copy(r1.at[rows], b)
    _add_rows(a, b, _RCH, 8)
    pltpu.sync_copy(a, out_r.at[rows])


_sc_rel_add = pl.kernel(
    _sc_rel_body,
    out_type=jax.ShapeDtypeStruct((_N_REL, _D), jnp.float32),
    mesh=plsc.VectorSubcoreMesh(
        core_axis_name="c",
        subcore_axis_name="s",
        num_cores=_NUM_CORES,
        num_subcores=_NUM_SUBCORES,
    ),
    scratch_types=[
        pltpu.VMEM((_RCH, _D), jnp.float32),
        pltpu.VMEM((_RCH, _D), jnp.float32),
    ],
)


_BLK = 5000


def _tc_both_body(a, b, r0, r1, o, orel):
    o[...] = a[...] + b[...]

    @pl.when(pl.program_id(0) == 0)
    def _():
        orel[...] = r0[...] + r1[...]


_tc_both_add = pl.pallas_call(
    _tc_both_body,
    grid=(_N_ENT // _BLK,),
    in_specs=[
        pl.BlockSpec((_BLK, _D), lambda i: (i, 0)),
        pl.BlockSpec((_BLK, _D), lambda i: (i, 0)),
        pl.BlockSpec((_N_REL, _D), lambda i: (0, 0)),
        pl.BlockSpec((_N_REL, _D), lambda i: (0, 0)),
    ],
    out_specs=[
        pl.BlockSpec((_BLK, _D), lambda i: (i, 0)),
        pl.BlockSpec((_N_REL, _D), lambda i: (0, 0)),
    ],
    out_shape=[
        jax.ShapeDtypeStruct((_N_ENT, _D), jnp.float32),
        jax.ShapeDtypeStruct((_N_REL, _D), jnp.float32),
    ],
)


def kernel(inputs, ent_embeds_0, rel_embeds_0, ent_embeds_1, rel_embeds_1):
    del inputs
    out_ent, out_rel = _tc_both_add(
        ent_embeds_0, ent_embeds_1, rel_embeds_0, rel_embeds_1
    )
    return (out_ent, out_rel)
